# baseline (speedup 1.0000x reference)
import functools

import jax
import jax.numpy as jnp
from jax import lax
from jax.experimental import pallas as pl
from jax.experimental.pallas import tpu as pltpu
from jax.experimental.pallas import tpu_sc as plsc

BATCH = 16384
LABEL_DIM = 32
TEXT_DIM = 768

ROWS_PER_QUAD = 4
QUAD_DIM = ROWS_PER_QUAD * LABEL_DIM
NUM_QUADS = 1000000 // ROWS_PER_QUAD

NC = 2
NS = 16
NW = NC * NS
B_PER_W = BATCH // NW
GRP = 16
CHUNK = 128
NCHUNK = B_PER_W // CHUNK

_MESH = plsc.VectorSubcoreMesh(core_axis_name="c", subcore_axis_name="s")


@functools.partial(
    pl.kernel,
    mesh=_MESH,
    out_type=jax.ShapeDtypeStruct((BATCH, QUAD_DIM), jnp.float32),
    scratch_types=[
        pltpu.VMEM((B_PER_W,), jnp.int32),
        pltpu.VMEM((CHUNK,), jnp.int32),
        pltpu.VMEM((CHUNK, QUAD_DIM), jnp.float32),
        pltpu.SemaphoreType.DMA,
        pltpu.SemaphoreType.DMA,
    ],
)
def _sc_gather(table_hbm, idx_hbm, out_hbm, idx_v, tid_v, buf_v, sem_g, sem_o):
    wid = lax.axis_index("s") * NC + lax.axis_index("c")
    base = wid * B_PER_W
    pltpu.sync_copy(idx_hbm.at[pl.ds(base, B_PER_W)], idx_v)

    def one_chunk(c, _):
        c0 = c * CHUNK
        for g in range(CHUNK // GRP):
            tid_v[pl.ds(g * GRP, GRP)] = idx_v[pl.ds(c0 + g * GRP, GRP)] >> 2
        pltpu.async_copy(table_hbm.at[tid_v], buf_v, sem_g).wait()
        pltpu.async_copy(buf_v, out_hbm.at[pl.ds(base + c0, CHUNK)], sem_o).wait()
        return 0

    lax.fori_loop(0, NCHUNK, one_chunk, 0)


def _mm_body(quad_ref, oh_ref, w_ref, b_ref, out_ref):
    quads = quad_ref[...]
    oh = oh_ref[...]
    sel = oh[:, 0:1] * quads[:, 0 * LABEL_DIM:1 * LABEL_DIM]
    for k in range(1, ROWS_PER_QUAD):
        sel += oh[:, k:k + 1] * quads[:, k * LABEL_DIM:(k + 1) * LABEL_DIM]
    out_ref[...] = lax.dot_general(
        sel, w_ref[...],
        (((1,), (1,)), ((), ())),
        preferred_element_type=jnp.float32,
    ) + b_ref[...]


BM = 2048


def kernel(labels, label_emb, W, b):
    table2 = label_emb.reshape(NUM_QUADS, QUAD_DIM)
    quads = _sc_gather(table2, labels)
    onehot = jax.nn.one_hot(labels % ROWS_PER_QUAD, ROWS_PER_QUAD,
                            dtype=jnp.float32)
    b2d = b.reshape(1, TEXT_DIM)
    out = pl.pallas_call(
        _mm_body,
        grid=(BATCH // BM,),
        in_specs=[
            pl.BlockSpec((BM, QUAD_DIM), lambda i: (i, 0)),
            pl.BlockSpec((BM, ROWS_PER_QUAD), lambda i: (i, 0)),
            pl.BlockSpec((TEXT_DIM, LABEL_DIM), lambda i: (0, 0)),
            pl.BlockSpec((1, TEXT_DIM), lambda i: (0, 0)),
        ],
        out_specs=pl.BlockSpec((BM, TEXT_DIM), lambda i: (i, 0)),
        out_shape=jax.ShapeDtypeStruct((BATCH, TEXT_DIM), jnp.float32),
    )(quads, onehot, W, b2d)
    return out


# speedup vs baseline: 1.3513x; 1.3513x over previous
import functools

import jax
import jax.numpy as jnp
from jax import lax
from jax.experimental import pallas as pl
from jax.experimental.pallas import tpu as pltpu
from jax.experimental.pallas import tpu_sc as plsc

BATCH = 16384
LABEL_DIM = 32
TEXT_DIM = 768

NC = 2
NS = 16
NW = NC * NS

BM = 2048
NBLK = BATCH // BM
SC_BLK = 3
SC_ROWS = SC_BLK * BM
TC_BLK = NBLK - SC_BLK
B_PER_W = SC_ROWS // NW
GRP = 16
NGRP = B_PER_W // GRP

_MESH = plsc.VectorSubcoreMesh(core_axis_name="c", subcore_axis_name="s")


@functools.partial(
    pl.kernel,
    mesh=_MESH,
    out_type=jax.ShapeDtypeStruct((SC_ROWS, LABEL_DIM), jnp.float32),
    scratch_types=[
        pltpu.VMEM((B_PER_W,), jnp.int32),
        pltpu.SemaphoreType.DMA,
    ],
    compiler_params=pltpu.CompilerParams(needs_layout_passes=False),
)
def _sc_gather(table_hbm, idx_hbm, out_hbm, idx_v, sem):
    wid = lax.axis_index("s") * NC + lax.axis_index("c")
    base = wid * B_PER_W
    pltpu.sync_copy(idx_hbm.at[pl.ds(base, B_PER_W)], idx_v)
    lanes = lax.iota(jnp.int32, GRP)

    def one_group(g, _):
        v = idx_v[pl.ds(g * GRP, GRP)]
        for l in range(GRP):
            row = lax.reduce_sum_p.bind(
                jnp.where(lanes == l, v, 0), axes=(0,))
            pltpu.make_async_copy(
                table_hbm.at[pl.ds(row, 1)],
                out_hbm.at[pl.ds(base + g * GRP + l, 1)],
                sem,
            ).start()
        return 0

    lax.fori_loop(0, NGRP, one_group, 0)
    pltpu.make_async_copy(
        table_hbm.at[pl.ds(0, B_PER_W)],
        out_hbm.at[pl.ds(base, B_PER_W)],
        sem,
    ).wait()


def _issue_rows(labels_smem, table_hbm, emb_v, sem, blk):
    base = blk * BM

    def issue(j, _):
        row = labels_smem[base + j]
        pltpu.make_async_copy(
            table_hbm.at[pl.ds(row, 1)], emb_v.at[pl.ds(j, 1)], sem
        ).start()
        return 0

    lax.fori_loop(0, BM, issue, 0, unroll=8)


def _proj(emb, w_ref, b_ref):
    return lax.dot_general(
        emb, w_ref[...],
        (((1,), (1,)), ((), ())),
        preferred_element_type=jnp.float32,
    ) + b_ref[...]


def _tc_a_body(labels_smem, table_hbm, w_ref, b_ref, out_ref, emb_v, sem):
    i = pl.program_id(0)

    @pl.when(i == 0)
    def _prologue():
        _issue_rows(labels_smem, table_hbm, emb_v.at[0], sem.at[0], SC_BLK)

    @pl.when(i + 1 < TC_BLK)
    def _next():
        _issue_rows(labels_smem, table_hbm, emb_v.at[(i + 1) % 2],
                    sem.at[(i + 1) % 2], SC_BLK + i + 1)

    pltpu.make_async_copy(
        table_hbm.at[pl.ds(0, BM)], emb_v.at[i % 2], sem.at[i % 2]
    ).wait()
    out_ref[...] = _proj(emb_v[i % 2], w_ref, b_ref)


def _tc_b_body(out_a_ref, emb_ref, w_ref, b_ref, out_ref):
    out_ref[...] = _proj(emb_ref[...], w_ref, b_ref)


def kernel(labels, label_emb, W, b):
    b2d = b.reshape(1, TEXT_DIM)
    emb_sc = _sc_gather(label_emb, labels)

    grid_spec = pltpu.PrefetchScalarGridSpec(
        num_scalar_prefetch=1,
        grid=(TC_BLK,),
        in_specs=[
            pl.BlockSpec(memory_space=pl.ANY),
            pl.BlockSpec((TEXT_DIM, LABEL_DIM), lambda i, *_: (0, 0)),
            pl.BlockSpec((1, TEXT_DIM), lambda i, *_: (0, 0)),
        ],
        out_specs=pl.BlockSpec((BM, TEXT_DIM), lambda i, *_: (i + SC_BLK, 0)),
        scratch_shapes=[
            pltpu.VMEM((2, BM, LABEL_DIM), jnp.float32),
            pltpu.SemaphoreType.DMA((2,)),
        ],
    )
    out_a = pl.pallas_call(
        _tc_a_body,
        grid_spec=grid_spec,
        out_shape=jax.ShapeDtypeStruct((BATCH, TEXT_DIM), jnp.float32),
    )(labels, label_emb, W, b2d)

    out = pl.pallas_call(
        _tc_b_body,
        grid=(SC_BLK,),
        in_specs=[
            pl.BlockSpec(memory_space=pl.ANY),
            pl.BlockSpec((BM, LABEL_DIM), lambda i: (i, 0)),
            pl.BlockSpec((TEXT_DIM, LABEL_DIM), lambda i: (0, 0)),
            pl.BlockSpec((1, TEXT_DIM), lambda i: (0, 0)),
        ],
        out_specs=pl.BlockSpec((BM, TEXT_DIM), lambda i: (i, 0)),
        out_shape=jax.ShapeDtypeStruct((BATCH, TEXT_DIM), jnp.float32),
        input_output_aliases={0: 0},
    )(out_a, emb_sc, W, b2d)
    return out
